# Initial kernel scaffold; baseline (speedup 1.0000x reference)
import jax
import jax.numpy as jnp
from jax.experimental import pallas as pl


def kernel(x, edge_index, batch, W1, b1, W2, b2, W3, b3, W4, b4, Wc1, bc1, Wc2, bc2):
    raise NotImplementedError("write your pallas kernel here")


# speedup vs baseline: 1.8785x; 1.8785x over previous
import functools

import jax
import jax.numpy as jnp
from jax import lax
from jax.experimental import pallas as pl
from jax.experimental.pallas import tpu as pltpu
from jax.experimental.pallas import tpu_sc as plsc

N = 50000
E = 800000
G = 16
HID = 64
NC = 2
NS = 16
NW = NC * NS
NP = 1568
NPAD = NW * NP
EW = 25008
ECH = 256
CB = 128
DB = 2000
RB = 3136
EB2 = 2000


def _gather_add_body(t_hbm, src_hbm, dst_hbm, p_hbm, didx, sidx, rd, rs, ps, sem):
    wid = lax.axis_index("s") * NC + lax.axis_index("c")
    base = wid * EW

    def do_chunk(off, n):
        off = pl.multiple_of(off, 16)
        pltpu.sync_copy(dst_hbm.at[pl.ds(off, n)], didx.at[pl.ds(0, n)])
        pltpu.sync_copy(src_hbm.at[pl.ds(off, n)], sidx.at[pl.ds(0, n)])
        pltpu.async_copy(t_hbm.at[didx.at[pl.ds(0, n)]], rd.at[pl.ds(0, n)], sem).wait()
        pltpu.async_copy(t_hbm.at[sidx.at[pl.ds(0, n)]], rs.at[pl.ds(0, n)], sem).wait()

        def addpair(i, _):
            for half in range(2):
                row = i * 2 + half
                for k in range(HID // 16):
                    ps[i, pl.ds(half * HID + k * 16, 16)] = (
                        rd[row, pl.ds(k * 16, 16)]
                        + rs[row, pl.ds(HID + k * 16, 16)])
            return 0

        lax.fori_loop(0, n // 2, addpair, 0)
        po = pl.multiple_of(off // 2, 8)
        pltpu.sync_copy(ps.at[pl.ds(0, n // 2)], p_hbm.at[pl.ds(po, n // 2)])

    nchunks = jnp.where(wid == NW - 1, (E - (NW - 1) * EW - 176) // ECH,
                        (EW - 176) // ECH)

    def chunk_loop(c, _):
        do_chunk(base + c * ECH, ECH)
        return 0

    lax.fori_loop(0, nchunks, chunk_loop, 0)
    do_chunk(base + nchunks * ECH, 176)


def _gather_add(T, src, dst):
    return pl.kernel(
        _gather_add_body,
        out_type=jax.ShapeDtypeStruct((E // 2, 2 * HID), jnp.float32),
        mesh=plsc.VectorSubcoreMesh(core_axis_name="c", subcore_axis_name="s"),
        scratch_types=[
            pltpu.VMEM((ECH,), jnp.int32),
            pltpu.VMEM((ECH,), jnp.int32),
            pltpu.VMEM((ECH, 2 * HID), jnp.float32),
            pltpu.VMEM((ECH, 2 * HID), jnp.float32),
            pltpu.VMEM((ECH // 2, 2 * HID), jnp.float32),
            pltpu.SemaphoreType.DMA,
        ],
    )(T, src, dst)


def _segmax_xla(H2, dst):
    agg = jax.ops.segment_max(H2.reshape(E, HID), dst, num_segments=NPAD)
    return jnp.where(jnp.isneginf(agg), 0.0, agg)


def _nodeproj_body(h_ref, wd_ref, ws_ref, b_ref, t_ref):
    h = h_ref[...]
    a = jnp.dot(h, wd_ref[...], preferred_element_type=jnp.float32) + b_ref[...]
    b = jnp.dot(h, ws_ref[...], preferred_element_type=jnp.float32)
    t_ref[...] = jnp.concatenate([a, b], axis=1)


def _nodeproj(h, Wd, Ws, b):
    f = h.shape[1]
    return pl.pallas_call(
        _nodeproj_body,
        grid=(NPAD // RB,),
        in_specs=[
            pl.BlockSpec((RB, f), lambda i: (i, 0)),
            pl.BlockSpec((f, HID), lambda i: (0, 0)),
            pl.BlockSpec((f, HID), lambda i: (0, 0)),
            pl.BlockSpec((1, HID), lambda i: (0, 0)),
        ],
        out_specs=pl.BlockSpec((RB, 2 * HID), lambda i: (i, 0)),
        out_shape=jax.ShapeDtypeStruct((NPAD, 2 * HID), jnp.float32),
    )(h, Wd, Ws, b)


def _lrelu(x):
    return jnp.where(x >= 0, x, 0.2 * x)


def _edge_mlp_body(p_ref, w_ref, b_ref, h_ref):
    m = _lrelu(p_ref[...])
    for half in range(2):
        sl = slice(half * HID, (half + 1) * HID)
        h = jnp.dot(m[:, sl], w_ref[...], preferred_element_type=jnp.float32) + b_ref[...]
        h_ref[:, sl] = _lrelu(h)


def _edge_mlp(P2, W, b):
    return pl.pallas_call(
        _edge_mlp_body,
        grid=(E // 2 // EB2,),
        in_specs=[
            pl.BlockSpec((EB2, 2 * HID), lambda i: (i, 0)),
            pl.BlockSpec((HID, HID), lambda i: (0, 0)),
            pl.BlockSpec((1, HID), lambda i: (0, 0)),
        ],
        out_specs=pl.BlockSpec((EB2, 2 * HID), lambda i: (i, 0)),
        out_shape=jax.ShapeDtypeStruct((E // 2, 2 * HID), jnp.float32),
    )(P2, W, b)


def _pool_head_body(h_ref, bid_ref, wc1_ref, bc1_ref, wc2_ref, bc2_ref, out_ref,
                    s_sum, s_max, s_cnt):
    i = pl.program_id(0)

    @pl.when(i == 0)
    def _():
        s_sum[...] = jnp.zeros_like(s_sum)
        s_max[...] = jnp.full_like(s_max, -jnp.inf)
        s_cnt[...] = jnp.zeros_like(s_cnt)

    h = h_ref[...]
    bid = bid_ref[...]
    onehot = (bid == lax.broadcasted_iota(jnp.int32, (1, 32), 1)).astype(jnp.float32)
    s_sum[...] += lax.dot_general(onehot, h, (((0,), (0,)), ((), ())),
                                  preferred_element_type=jnp.float32)
    s_cnt[...] += jnp.sum(onehot, axis=0, keepdims=True)
    for g in range(G + 1):
        mg = jnp.max(jnp.where(bid == g, h, -jnp.inf), axis=0)
        s_max[g, :] = jnp.maximum(s_max[g, :], mg)

    @pl.when(i == pl.num_programs(0) - 1)
    def _():
        cnt = jnp.maximum(s_cnt[...], 1.0)
        mean = s_sum[...] / cnt.reshape(32, 1)
        mx = s_max[...]
        mx = jnp.where(mx == -jnp.inf, 0.0, mx)
        gfeat = jnp.concatenate([mean[:G, :], mx[:G, :]], axis=1)
        hc = _lrelu(jnp.dot(gfeat, wc1_ref[...], preferred_element_type=jnp.float32)
                    + bc1_ref[...])
        logits = jnp.sum(hc * wc2_ref[...], axis=1) + bc2_ref[0, 0]
        out_ref[...] = jnp.broadcast_to(logits[:, None], (G, 128))


def _pool_head(h, bid, Wc1, bc1, Wc2, bc2):
    return pl.pallas_call(
        _pool_head_body,
        grid=(NPAD // RB,),
        in_specs=[
            pl.BlockSpec((RB, HID), lambda i: (i, 0)),
            pl.BlockSpec((RB, 1), lambda i: (i, 0)),
            pl.BlockSpec((2 * HID, HID), lambda i: (0, 0)),
            pl.BlockSpec((1, HID), lambda i: (0, 0)),
            pl.BlockSpec((1, HID), lambda i: (0, 0)),
            pl.BlockSpec((1, 1), lambda i: (0, 0)),
        ],
        out_specs=pl.BlockSpec((G, 128), lambda i: (0, 0)),
        out_shape=jax.ShapeDtypeStruct((G, 128), jnp.float32),
        scratch_shapes=[
            pltpu.VMEM((32, HID), jnp.float32),
            pltpu.VMEM((32, HID), jnp.float32),
            pltpu.VMEM((1, 32), jnp.float32),
        ],
    )(h, bid, Wc1, bc1, Wc2, bc2)


def kernel(x, edge_index, batch, W1, b1, W2, b2, W3, b3, W4, b4, Wc1, bc1, Wc2, bc2):
    src = edge_index[0]
    dst = edge_index[1]

    xp = jnp.zeros((NPAD, 8), jnp.float32).at[:N, :6].set(x)
    Wd1 = jnp.zeros((8, HID), jnp.float32).at[:6].set(W1[:6] - W1[6:])
    Ws1 = jnp.zeros((8, HID), jnp.float32).at[:6].set(W1[6:])

    T1 = _nodeproj(xp, Wd1, Ws1, b1.reshape(1, HID))
    P1 = _gather_add(T1, src, dst)
    H1 = _edge_mlp(P1, W2, b2.reshape(1, HID))
    h1 = _segmax_xla(H1, dst)

    Wd2 = W3[:HID] - W3[HID:]
    Ws2 = W3[HID:]
    T2 = _nodeproj(h1, Wd2, Ws2, b3.reshape(1, HID))
    P2 = _gather_add(T2, src, dst)
    H2 = _edge_mlp(P2, W4, b4.reshape(1, HID))
    h2 = _segmax_xla(H2, dst)

    bid = jnp.concatenate([batch, jnp.full((NPAD - N,), G, jnp.int32)]).reshape(NPAD, 1)
    out = _pool_head(h2, bid, Wc1, bc1.reshape(1, HID), Wc2.reshape(1, HID),
                     bc2.reshape(1, 1))
    return out[:, 0]


# speedup vs baseline: 2.0546x; 1.0937x over previous
import functools

import jax
import jax.numpy as jnp
from jax import lax
from jax.experimental import pallas as pl
from jax.experimental.pallas import tpu as pltpu
from jax.experimental.pallas import tpu_sc as plsc

N = 50000
E = 800000
G = 16
HID = 64
NC = 2
NS = 16
NW = NC * NS
NP = 1568
NPAD = NW * NP
EW = 25008
ECH = 128
CB = 128
DB = 2000
RB = 3136
EB2 = 2000


def _gather_add_body(t_hbm, src_hbm, dst_hbm, p_hbm,
                     didx0, sidx0, rd0, rs0, ps0,
                     didx1, sidx1, rd1, rs1, ps1, gsem, wsem):
    wid = lax.axis_index("s") * NC + lax.axis_index("c")
    base = wid * EW
    bufs = ((didx0, sidx0, rd0, rs0, ps0), (didx1, sidx1, rd1, rs1, ps1))

    def fire(b, off, n):
        didx, sidx, rd, rs, _ = bufs[b]
        off = pl.multiple_of(off, 16)
        pltpu.sync_copy(dst_hbm.at[pl.ds(off, n)], didx.at[pl.ds(0, n)])
        pltpu.sync_copy(src_hbm.at[pl.ds(off, n)], sidx.at[pl.ds(0, n)])
        d1 = pltpu.async_copy(t_hbm.at[didx.at[pl.ds(0, n)]], rd.at[pl.ds(0, n)], gsem)
        d2 = pltpu.async_copy(t_hbm.at[sidx.at[pl.ds(0, n)]], rs.at[pl.ds(0, n)], gsem)
        return d1, d2

    def drain(b, n):
        didx, sidx, rd, rs, _ = bufs[b]
        pltpu.make_async_copy(t_hbm.at[didx.at[pl.ds(0, n)]], rd.at[pl.ds(0, n)], gsem).wait()
        pltpu.make_async_copy(t_hbm.at[sidx.at[pl.ds(0, n)]], rs.at[pl.ds(0, n)], gsem).wait()

    def process(b, off, n, drain_write):
        didx, sidx, rd, rs, ps = bufs[b]
        off = pl.multiple_of(off, 16)
        po = pl.multiple_of(off // 2, 8)

        def maybe_drain(_):
            pltpu.make_async_copy(ps.at[pl.ds(0, n // 2)],
                                  p_hbm.at[pl.ds(po, n // 2)], wsem).wait()
            return 0

        lax.cond(drain_write, maybe_drain, lambda _: 0, 0)

        def addpair(i, _):
            for half in range(2):
                row = i * 2 + half
                for k in range(HID // 16):
                    ps[i, pl.ds(half * HID + k * 16, 16)] = (
                        rd[row, pl.ds(k * 16, 16)]
                        + rs[row, pl.ds(HID + k * 16, 16)])
            return 0

        lax.fori_loop(0, n // 2, addpair, 0)
        pltpu.async_copy(ps.at[pl.ds(0, n // 2)], p_hbm.at[pl.ds(po, n // 2)], wsem)

    nchunks = jnp.where(wid == NW - 1, (E - (NW - 1) * EW - 48) // ECH,
                        (EW - 48) // ECH)
    toff = base + nchunks * ECH
    fire(0, toff, 48)
    drain(0, 48)
    process(0, toff, 48, False)
    pltpu.make_async_copy(ps0.at[pl.ds(0, 24)],
                          p_hbm.at[pl.ds(0, 24)], wsem).wait()

    fire(0, base, ECH)

    def pipe(i, _):
        c0 = base + (2 * i) * ECH
        c1 = base + (2 * i + 1) * ECH
        c2 = base + (2 * i + 2) * ECH
        dA = fire(1, c1, ECH)
        drain(0, ECH)
        process(0, c0, ECH, i >= 1)
        fire(0, c2, ECH)
        dA[0].wait()
        dA[1].wait()
        process(1, c1, ECH, i >= 1)
        return 0

    lax.fori_loop(0, (nchunks - 1) // 2, pipe, 0)
    last = base + (nchunks - 1) * ECH
    drain(0, ECH)
    process(0, last, ECH, True)
    pltpu.make_async_copy(ps0.at[pl.ds(0, ECH // 2)],
                          p_hbm.at[pl.ds(0, ECH // 2)], wsem).wait()
    pltpu.make_async_copy(ps1.at[pl.ds(0, ECH // 2)],
                          p_hbm.at[pl.ds(0, ECH // 2)], wsem).wait()


def _gather_add(T, src, dst):
    return pl.kernel(
        _gather_add_body,
        out_type=jax.ShapeDtypeStruct((E // 2, 2 * HID), jnp.float32),
        mesh=plsc.VectorSubcoreMesh(core_axis_name="c", subcore_axis_name="s"),
        scratch_types=[
            pltpu.VMEM((ECH,), jnp.int32),
            pltpu.VMEM((ECH,), jnp.int32),
            pltpu.VMEM((ECH, 2 * HID), jnp.float32),
            pltpu.VMEM((ECH, 2 * HID), jnp.float32),
            pltpu.VMEM((ECH // 2, 2 * HID), jnp.float32),
            pltpu.VMEM((ECH,), jnp.int32),
            pltpu.VMEM((ECH,), jnp.int32),
            pltpu.VMEM((ECH, 2 * HID), jnp.float32),
            pltpu.VMEM((ECH, 2 * HID), jnp.float32),
            pltpu.VMEM((ECH // 2, 2 * HID), jnp.float32),
            pltpu.SemaphoreType.DMA,
            pltpu.SemaphoreType.DMA,
        ],
    )(T, src, dst)


def _segmax_xla(H2, dst):
    agg = jax.ops.segment_max(H2.reshape(E, HID), dst, num_segments=NPAD)
    return jnp.where(jnp.isneginf(agg), 0.0, agg)


def _nodeproj_body(h_ref, wd_ref, ws_ref, b_ref, t_ref):
    h = h_ref[...]
    a = jnp.dot(h, wd_ref[...], preferred_element_type=jnp.float32) + b_ref[...]
    b = jnp.dot(h, ws_ref[...], preferred_element_type=jnp.float32)
    t_ref[...] = jnp.concatenate([a, b], axis=1)


def _nodeproj(h, Wd, Ws, b):
    f = h.shape[1]
    return pl.pallas_call(
        _nodeproj_body,
        grid=(NPAD // RB,),
        in_specs=[
            pl.BlockSpec((RB, f), lambda i: (i, 0)),
            pl.BlockSpec((f, HID), lambda i: (0, 0)),
            pl.BlockSpec((f, HID), lambda i: (0, 0)),
            pl.BlockSpec((1, HID), lambda i: (0, 0)),
        ],
        out_specs=pl.BlockSpec((RB, 2 * HID), lambda i: (i, 0)),
        out_shape=jax.ShapeDtypeStruct((NPAD, 2 * HID), jnp.float32),
    )(h, Wd, Ws, b)


def _lrelu(x):
    return jnp.where(x >= 0, x, 0.2 * x)


def _edge_mlp_body(p_ref, w_ref, b_ref, h_ref):
    m = _lrelu(p_ref[...])
    for half in range(2):
        sl = slice(half * HID, (half + 1) * HID)
        h = jnp.dot(m[:, sl], w_ref[...], preferred_element_type=jnp.float32) + b_ref[...]
        h_ref[:, sl] = _lrelu(h)


def _edge_mlp(P2, W, b):
    return pl.pallas_call(
        _edge_mlp_body,
        grid=(E // 2 // EB2,),
        in_specs=[
            pl.BlockSpec((EB2, 2 * HID), lambda i: (i, 0)),
            pl.BlockSpec((HID, HID), lambda i: (0, 0)),
            pl.BlockSpec((1, HID), lambda i: (0, 0)),
        ],
        out_specs=pl.BlockSpec((EB2, 2 * HID), lambda i: (i, 0)),
        out_shape=jax.ShapeDtypeStruct((E // 2, 2 * HID), jnp.float32),
    )(P2, W, b)


def _pool_head_body(h_ref, bid_ref, wc1_ref, bc1_ref, wc2_ref, bc2_ref, out_ref,
                    s_sum, s_max, s_cnt):
    i = pl.program_id(0)

    @pl.when(i == 0)
    def _():
        s_sum[...] = jnp.zeros_like(s_sum)
        s_max[...] = jnp.full_like(s_max, -jnp.inf)
        s_cnt[...] = jnp.zeros_like(s_cnt)

    h = h_ref[...]
    bid = bid_ref[...]
    onehot = (bid == lax.broadcasted_iota(jnp.int32, (1, 32), 1)).astype(jnp.float32)
    s_sum[...] += lax.dot_general(onehot, h, (((0,), (0,)), ((), ())),
                                  preferred_element_type=jnp.float32)
    s_cnt[...] += jnp.sum(onehot, axis=0, keepdims=True)
    for g in range(G + 1):
        mg = jnp.max(jnp.where(bid == g, h, -jnp.inf), axis=0)
        s_max[g, :] = jnp.maximum(s_max[g, :], mg)

    @pl.when(i == pl.num_programs(0) - 1)
    def _():
        cnt = jnp.maximum(s_cnt[...], 1.0)
        mean = s_sum[...] / cnt.reshape(32, 1)
        mx = s_max[...]
        mx = jnp.where(mx == -jnp.inf, 0.0, mx)
        gfeat = jnp.concatenate([mean[:G, :], mx[:G, :]], axis=1)
        hc = _lrelu(jnp.dot(gfeat, wc1_ref[...], preferred_element_type=jnp.float32)
                    + bc1_ref[...])
        logits = jnp.sum(hc * wc2_ref[...], axis=1) + bc2_ref[0, 0]
        out_ref[...] = jnp.broadcast_to(logits[:, None], (G, 128))


def _pool_head(h, bid, Wc1, bc1, Wc2, bc2):
    return pl.pallas_call(
        _pool_head_body,
        grid=(NPAD // RB,),
        in_specs=[
            pl.BlockSpec((RB, HID), lambda i: (i, 0)),
            pl.BlockSpec((RB, 1), lambda i: (i, 0)),
            pl.BlockSpec((2 * HID, HID), lambda i: (0, 0)),
            pl.BlockSpec((1, HID), lambda i: (0, 0)),
            pl.BlockSpec((1, HID), lambda i: (0, 0)),
            pl.BlockSpec((1, 1), lambda i: (0, 0)),
        ],
        out_specs=pl.BlockSpec((G, 128), lambda i: (0, 0)),
        out_shape=jax.ShapeDtypeStruct((G, 128), jnp.float32),
        scratch_shapes=[
            pltpu.VMEM((32, HID), jnp.float32),
            pltpu.VMEM((32, HID), jnp.float32),
            pltpu.VMEM((1, 32), jnp.float32),
        ],
    )(h, bid, Wc1, bc1, Wc2, bc2)


def kernel(x, edge_index, batch, W1, b1, W2, b2, W3, b3, W4, b4, Wc1, bc1, Wc2, bc2):
    src = edge_index[0]
    dst = edge_index[1]

    xp = jnp.zeros((NPAD, 8), jnp.float32).at[:N, :6].set(x)
    Wd1 = jnp.zeros((8, HID), jnp.float32).at[:6].set(W1[:6] - W1[6:])
    Ws1 = jnp.zeros((8, HID), jnp.float32).at[:6].set(W1[6:])

    T1 = _nodeproj(xp, Wd1, Ws1, b1.reshape(1, HID))
    P1 = _gather_add(T1, src, dst)
    H1 = _edge_mlp(P1, W2, b2.reshape(1, HID))
    h1 = _segmax_xla(H1, dst)

    Wd2 = W3[:HID] - W3[HID:]
    Ws2 = W3[HID:]
    T2 = _nodeproj(h1, Wd2, Ws2, b3.reshape(1, HID))
    P2 = _gather_add(T2, src, dst)
    H2 = _edge_mlp(P2, W4, b4.reshape(1, HID))
    h2 = _segmax_xla(H2, dst)

    bid = jnp.concatenate([batch, jnp.full((NPAD - N,), G, jnp.int32)]).reshape(NPAD, 1)
    out = _pool_head(h2, bid, Wc1, bc1.reshape(1, HID), Wc2.reshape(1, HID),
                     bc2.reshape(1, 1))
    return out[:, 0]


# speedup vs baseline: 2.0551x; 1.0002x over previous
import jax
import jax.numpy as jnp
from jax import lax
from jax.experimental import pallas as pl
from jax.experimental.pallas import tpu as pltpu
from jax.experimental.pallas import tpu_sc as plsc

N = 50000
E = 800000
G = 16
HID = 64
NC = 2
NS = 16
NW = NC * NS
NP = 1568
NPAD = NW * NP
EW = 25008
ECH = 128
CB = 128
DB = 2000
RB = 3136
EB2 = 2000


def _gather_add_body(t_hbm, src_hbm, dst_hbm, p_hbm,
                     didx0, sidx0, rd0, rs0, ps0,
                     didx1, sidx1, rd1, rs1, ps1,
                     gsem0, wsem0, gsem1, wsem1):
    wid = lax.axis_index("s") * NC + lax.axis_index("c")
    base = wid * EW
    bufs = ((didx0, sidx0, rd0, rs0, ps0, gsem0, wsem0),
            (didx1, sidx1, rd1, rs1, ps1, gsem1, wsem1))

    def fire(b, off, n):
        didx, sidx, rd, rs, _, gsem, _w = bufs[b]
        off = pl.multiple_of(off, 16)
        pltpu.sync_copy(dst_hbm.at[pl.ds(off, n)], didx.at[pl.ds(0, n)])
        pltpu.sync_copy(src_hbm.at[pl.ds(off, n)], sidx.at[pl.ds(0, n)])
        d1 = pltpu.async_copy(t_hbm.at[didx.at[pl.ds(0, n)]], rd.at[pl.ds(0, n)], gsem)
        d2 = pltpu.async_copy(t_hbm.at[sidx.at[pl.ds(0, n)]], rs.at[pl.ds(0, n)], gsem)
        return d1, d2

    def drain(b, n):
        didx, sidx, rd, rs, _, gsem, _w = bufs[b]
        pltpu.make_async_copy(t_hbm.at[didx.at[pl.ds(0, n)]], rd.at[pl.ds(0, n)], gsem).wait()
        pltpu.make_async_copy(t_hbm.at[sidx.at[pl.ds(0, n)]], rs.at[pl.ds(0, n)], gsem).wait()

    def process(b, off, n, drain_write):
        didx, sidx, rd, rs, ps, _g, wsem = bufs[b]
        off = pl.multiple_of(off, 16)
        po = pl.multiple_of(off // 2, 8)

        def maybe_drain(_):
            pltpu.make_async_copy(ps.at[pl.ds(0, n // 2)],
                                  p_hbm.at[pl.ds(po, n // 2)], wsem).wait()
            return 0

        lax.cond(drain_write, maybe_drain, lambda _: 0, 0)

        def addpair(i, _):
            for half in range(2):
                row = i * 2 + half
                for k in range(HID // 16):
                    ps[i, pl.ds(half * HID + k * 16, 16)] = (
                        rd[row, pl.ds(k * 16, 16)]
                        + rs[row, pl.ds(HID + k * 16, 16)])
            return 0

        lax.fori_loop(0, n // 2, addpair, 0)
        pltpu.async_copy(ps.at[pl.ds(0, n // 2)], p_hbm.at[pl.ds(po, n // 2)], wsem)

    nchunks = jnp.where(wid == NW - 1, (E - (NW - 1) * EW - 48) // ECH,
                        (EW - 48) // ECH)
    toff = base + nchunks * ECH
    fire(0, toff, 48)
    drain(0, 48)
    process(0, toff, 48, False)
    pltpu.make_async_copy(ps0.at[pl.ds(0, 24)],
                          p_hbm.at[pl.ds(0, 24)], wsem0).wait()

    fire(0, base, ECH)

    def pipe(i, _):
        c0 = base + (2 * i) * ECH
        c1 = base + (2 * i + 1) * ECH
        c2 = base + (2 * i + 2) * ECH
        dA = fire(1, c1, ECH)
        drain(0, ECH)
        process(0, c0, ECH, i >= 1)
        fire(0, c2, ECH)
        dA[0].wait()
        dA[1].wait()
        process(1, c1, ECH, i >= 1)
        return 0

    lax.fori_loop(0, (nchunks - 1) // 2, pipe, 0)
    last = base + (nchunks - 1) * ECH
    drain(0, ECH)
    process(0, last, ECH, True)
    pltpu.make_async_copy(ps0.at[pl.ds(0, ECH // 2)],
                          p_hbm.at[pl.ds(0, ECH // 2)], wsem0).wait()
    pltpu.make_async_copy(ps1.at[pl.ds(0, ECH // 2)],
                          p_hbm.at[pl.ds(0, ECH // 2)], wsem1).wait()


def _gather_add(T, src, dst):
    return pl.kernel(
        _gather_add_body,
        out_type=jax.ShapeDtypeStruct((E // 2, 2 * HID), jnp.float32),
        mesh=plsc.VectorSubcoreMesh(core_axis_name="c", subcore_axis_name="s"),
        scratch_types=[
            pltpu.VMEM((ECH,), jnp.int32),
            pltpu.VMEM((ECH,), jnp.int32),
            pltpu.VMEM((ECH, 2 * HID), jnp.float32),
            pltpu.VMEM((ECH, 2 * HID), jnp.float32),
            pltpu.VMEM((ECH // 2, 2 * HID), jnp.float32),
            pltpu.VMEM((ECH,), jnp.int32),
            pltpu.VMEM((ECH,), jnp.int32),
            pltpu.VMEM((ECH, 2 * HID), jnp.float32),
            pltpu.VMEM((ECH, 2 * HID), jnp.float32),
            pltpu.VMEM((ECH // 2, 2 * HID), jnp.float32),
            pltpu.SemaphoreType.DMA,
            pltpu.SemaphoreType.DMA,
            pltpu.SemaphoreType.DMA,
            pltpu.SemaphoreType.DMA,
        ],
    )(T, src, dst)


def _segmax_xla(H2, dst):
    agg = jax.ops.segment_max(H2.reshape(E, HID), dst, num_segments=NPAD)
    return jnp.where(jnp.isneginf(agg), 0.0, agg)


def _nodeproj_body(h_ref, wd_ref, ws_ref, b_ref, t_ref):
    h = h_ref[...]
    a = jnp.dot(h, wd_ref[...], preferred_element_type=jnp.float32) + b_ref[...]
    b = jnp.dot(h, ws_ref[...], preferred_element_type=jnp.float32)
    t_ref[...] = jnp.concatenate([a, b], axis=1)


def _nodeproj(h, Wd, Ws, b):
    f = h.shape[1]
    return pl.pallas_call(
        _nodeproj_body,
        grid=(NPAD // RB,),
        in_specs=[
            pl.BlockSpec((RB, f), lambda i: (i, 0)),
            pl.BlockSpec((f, HID), lambda i: (0, 0)),
            pl.BlockSpec((f, HID), lambda i: (0, 0)),
            pl.BlockSpec((1, HID), lambda i: (0, 0)),
        ],
        out_specs=pl.BlockSpec((RB, 2 * HID), lambda i: (i, 0)),
        out_shape=jax.ShapeDtypeStruct((NPAD, 2 * HID), jnp.float32),
    )(h, Wd, Ws, b)


def _lrelu(x):
    return jnp.where(x >= 0, x, 0.2 * x)


def _edge_mlp_body(p_ref, w_ref, b_ref, h_ref):
    m = _lrelu(p_ref[...])
    for half in range(2):
        sl = slice(half * HID, (half + 1) * HID)
        h = jnp.dot(m[:, sl], w_ref[...], preferred_element_type=jnp.float32) + b_ref[...]
        h_ref[:, sl] = _lrelu(h)


def _edge_mlp(P2, W, b):
    return pl.pallas_call(
        _edge_mlp_body,
        grid=(E // 2 // EB2,),
        in_specs=[
            pl.BlockSpec((EB2, 2 * HID), lambda i: (i, 0)),
            pl.BlockSpec((HID, HID), lambda i: (0, 0)),
            pl.BlockSpec((1, HID), lambda i: (0, 0)),
        ],
        out_specs=pl.BlockSpec((EB2, 2 * HID), lambda i: (i, 0)),
        out_shape=jax.ShapeDtypeStruct((E // 2, 2 * HID), jnp.float32),
    )(P2, W, b)


def _pool_head_body(h_ref, bid_ref, wc1_ref, bc1_ref, wc2_ref, bc2_ref, out_ref,
                    s_sum, s_max, s_cnt):
    i = pl.program_id(0)

    @pl.when(i == 0)
    def _():
        s_sum[...] = jnp.zeros_like(s_sum)
        s_max[...] = jnp.full_like(s_max, -jnp.inf)
        s_cnt[...] = jnp.zeros_like(s_cnt)

    h = h_ref[...]
    bid = bid_ref[...]
    onehot = (bid == lax.broadcasted_iota(jnp.int32, (1, 32), 1)).astype(jnp.float32)
    s_sum[...] += lax.dot_general(onehot, h, (((0,), (0,)), ((), ())),
                                  preferred_element_type=jnp.float32)
    s_cnt[...] += jnp.sum(onehot, axis=0, keepdims=True)
    for g in range(G + 1):
        mg = jnp.max(jnp.where(bid == g, h, -jnp.inf), axis=0)
        s_max[g, :] = jnp.maximum(s_max[g, :], mg)

    @pl.when(i == pl.num_programs(0) - 1)
    def _():
        cnt = jnp.maximum(s_cnt[...], 1.0)
        mean = s_sum[...] / cnt.reshape(32, 1)
        mx = s_max[...]
        mx = jnp.where(mx == -jnp.inf, 0.0, mx)
        gfeat = jnp.concatenate([mean[:G, :], mx[:G, :]], axis=1)
        hc = _lrelu(jnp.dot(gfeat, wc1_ref[...], preferred_element_type=jnp.float32)
                    + bc1_ref[...])
        logits = jnp.sum(hc * wc2_ref[...], axis=1) + bc2_ref[0, 0]
        out_ref[...] = jnp.broadcast_to(logits[:, None], (G, 128))


def _pool_head(h, bid, Wc1, bc1, Wc2, bc2):
    return pl.pallas_call(
        _pool_head_body,
        grid=(NPAD // RB,),
        in_specs=[
            pl.BlockSpec((RB, HID), lambda i: (i, 0)),
            pl.BlockSpec((RB, 1), lambda i: (i, 0)),
            pl.BlockSpec((2 * HID, HID), lambda i: (0, 0)),
            pl.BlockSpec((1, HID), lambda i: (0, 0)),
            pl.BlockSpec((1, HID), lambda i: (0, 0)),
            pl.BlockSpec((1, 1), lambda i: (0, 0)),
        ],
        out_specs=pl.BlockSpec((G, 128), lambda i: (0, 0)),
        out_shape=jax.ShapeDtypeStruct((G, 128), jnp.float32),
        scratch_shapes=[
            pltpu.VMEM((32, HID), jnp.float32),
            pltpu.VMEM((32, HID), jnp.float32),
            pltpu.VMEM((1, 32), jnp.float32),
        ],
    )(h, bid, Wc1, bc1, Wc2, bc2)


def kernel(x, edge_index, batch, W1, b1, W2, b2, W3, b3, W4, b4, Wc1, bc1, Wc2, bc2):
    src = edge_index[0]
    dst = edge_index[1]

    xp = jnp.zeros((NPAD, 8), jnp.float32).at[:N, :6].set(x)
    Wd1 = jnp.zeros((8, HID), jnp.float32).at[:6].set(W1[:6] - W1[6:])
    Ws1 = jnp.zeros((8, HID), jnp.float32).at[:6].set(W1[6:])

    T1 = _nodeproj(xp, Wd1, Ws1, b1.reshape(1, HID))
    P1 = _gather_add(T1, src, dst)
    H1 = _edge_mlp(P1, W2, b2.reshape(1, HID))
    h1 = _segmax_xla(H1, dst)

    Wd2 = W3[:HID] - W3[HID:]
    Ws2 = W3[HID:]
    T2 = _nodeproj(h1, Wd2, Ws2, b3.reshape(1, HID))
    P2 = _gather_add(T2, src, dst)
    H2 = _edge_mlp(P2, W4, b4.reshape(1, HID))
    h2 = _segmax_xla(H2, dst)

    bid = jnp.concatenate([batch, jnp.full((NPAD - N,), G, jnp.int32)]).reshape(NPAD, 1)
    out = _pool_head(h2, bid, Wc1, bc1.reshape(1, HID), Wc2.reshape(1, HID),
                     bc2.reshape(1, 1))
    return out[:, 0]
